# Initial kernel scaffold; baseline (speedup 1.0000x reference)
import jax
import jax.numpy as jnp
from jax.experimental import pallas as pl


def kernel(x, W1, b1, W2, b2):
    raise NotImplementedError("write your pallas kernel here")


# speedup vs baseline: 1.3477x; 1.3477x over previous
import functools

import jax
import jax.numpy as jnp
from jax import lax
from jax.experimental import pallas as pl
from jax.experimental.pallas import tpu as pltpu
from jax.experimental.pallas import tpu_sc as plsc

B, S, H, D = 4, 8192, 1024, 256
K = S // 2
RB = 1024
NROWS = B * S
NS = 16
TOK = S // NS
OUTT = K // NS
IDXPAD = 4224

_SQRT_HALF = 0.7071067811865476
_MAXLOG = 88.72283905206835
_ERFC_P = (2.326819970068386e-2, -1.387039388740657e-1, 3.687424674597105e-1,
           -5.824733027278666e-1, 6.210004621745983e-1, -4.944515323274145e-1,
           3.404879937665872e-1, -2.741127028184656e-1, 5.638259427386472e-1)
_ERFC_R = (-1.047766399936249e+1, 1.297719955372516e+1, -7.495518717768503e+0,
           2.921019019210786e+0, -1.015265279202700e+0, 4.218463358204948e-1,
           -2.820767439740514e-1, 5.641895067754075e-1)
_ERF_T = (7.853861353153693e-5, -8.010193625184903e-4, 5.188327685732524e-3,
          -2.685381193529856e-2, 1.128358514861418e-1, -3.761262582423300e-1,
          1.128379165726710e+0)


def _poly(y, coeffs):
    p = jnp.zeros_like(y)
    for c in coeffs:
        p = p * y + jnp.float32(c)
    return p


def _erfc_f32(x):
    one, two, zero = jnp.float32(1), jnp.float32(2), jnp.float32(0)
    x2 = x * x
    z = -x2
    absx = jnp.abs(x)
    rabs = one / absx
    y = rabs * rabs
    ez = jnp.exp(z)
    poll = jnp.where(absx < two, _poly(y, _ERFC_P), _poly(y, _ERFC_R))
    erfc_ge = (ez * rabs) * poll
    erfc_ge = jnp.where(z < jnp.float32(-_MAXLOG), zero, erfc_ge)
    erfc_ge = jnp.where(x < zero, two - erfc_ge, erfc_ge)
    erf_small = x * _poly(x2, _ERF_T)
    return jnp.where(absx < one, one - erf_small, erfc_ge)


def _score_body(x_ref, w1_ref, b1_ref, w2_ref, b2_ref, out_ref):
    xb = x_ref[...]
    h_pre = lax.dot_general(
        xb, w1_ref[...], (((1,), (0,)), ((), ())),
        preferred_element_type=jnp.float32)
    h_pre = h_pre + b1_ref[...]
    g = (jnp.float32(0.5) * h_pre) * _erfc_f32(-h_pre * jnp.float32(_SQRT_HALF))
    imp = lax.dot_general(
        g, w2_ref[...], (((1,), (0,)), ((), ())),
        preferred_element_type=jnp.float32) + b2_ref[0, 0]
    out_ref[...] = imp


def _scores(x2d, W1, b1, W2, b2):
    return pl.pallas_call(
        _score_body,
        grid=(NROWS // RB,),
        in_specs=[
            pl.BlockSpec((RB, H), lambda i: (i, 0)),
            pl.BlockSpec((H, D), lambda i: (0, 0)),
            pl.BlockSpec((1, D), lambda i: (0, 0)),
            pl.BlockSpec((D, 1), lambda i: (0, 0)),
            pl.BlockSpec(memory_space=pltpu.SMEM),
        ],
        out_specs=pl.BlockSpec((RB, 1), lambda i: (i, 0)),
        out_shape=jax.ShapeDtypeStruct((NROWS, 1), jnp.float32),
        compiler_params=pltpu.CompilerParams(
            vmem_limit_bytes=100 * 1024 * 1024),
    )(x2d, W1, b1.reshape(1, D), W2, b2.reshape(1, 1))


def _sortkey(b):
    return b ^ ((b >> 31) & jnp.int32(0x7FFFFFFF))


def _thr_body(s_ref, out_ref):
    keys = _sortkey(lax.bitcast_convert_type(s_ref[...], jnp.int32))
    kk = jnp.int32(K)
    isign = jnp.int32(-2147483648)

    def body(t, vu):
        bit = 31 - t
        cand_u = vu | (jnp.int32(1) << bit)
        cand_s = cand_u ^ isign
        cnt = jnp.sum((keys >= cand_s).astype(jnp.int32), axis=1, keepdims=True)
        return jnp.where(cnt >= kk, cand_u, vu)

    vu = lax.fori_loop(0, 32, body, jnp.zeros((B, 1), jnp.int32))
    vs = vu ^ isign
    out_ref[...] = jnp.broadcast_to(vs, (B, 128))


def _threshold(scores4):
    return pl.pallas_call(
        _thr_body,
        in_specs=[pl.BlockSpec((B, S), lambda: (0, 0))],
        out_specs=pl.BlockSpec((B, 128), lambda: (0, 0)),
        out_shape=jax.ShapeDtypeStruct((B, 128), jnp.int32),
    )(scores4)


def _lane(v, i, iota):
    return jnp.sum(jnp.where(iota == i, v, 0))


def _sc_body(x_hbm, sc_hbm, thr_hbm, out_hbm,
             sbuf, thrv, cbuf, cntrow, pos2d, vals2d, idxg, rowbuf,
             idx_sh, cnt_sh, sem, semg):
    c = lax.axis_index("c")
    s = lax.axis_index("s")
    iota = lax.iota(jnp.int32, 16)
    pltpu.sync_copy(thr_hbm, thrv)
    thrvec = thrv[...]

    cnts = []
    for lb in range(2):
        gb = 2 * c + lb
        off = gb * S + s * TOK
        pltpu.sync_copy(sc_hbm.at[pl.ds(off, TOK)], sbuf.at[pl.ds(lb * TOK, TOK)])
        thr_b = _lane(thrvec, gb, iota)
        gcnt = jnp.int32(0)
        ecnt = jnp.int32(0)
        for ch in range(TOK // 16):
            sv = sbuf[pl.ds(lb * TOK + ch * 16, 16)]
            kv = _sortkey(plsc.bitcast(sv, jnp.int32))
            gcnt = gcnt + jnp.sum((kv > thr_b).astype(jnp.int32))
            ecnt = ecnt + jnp.sum((kv == thr_b).astype(jnp.int32))
        cnts += [gcnt, ecnt]
    row = (jnp.where(iota == 0, cnts[0], 0) + jnp.where(iota == 1, cnts[1], 0)
           + jnp.where(iota == 2, cnts[2], 0) + jnp.where(iota == 3, cnts[3], 0))
    cntrow[...] = row
    pltpu.sync_copy(cntrow, cnt_sh.at[pl.ds(s * 16, 16)])
    plsc.subcore_barrier()

    pltpu.sync_copy(cnt_sh, cbuf)
    pref = jnp.zeros((16,), jnp.int32)
    tot = jnp.zeros((16,), jnp.int32)
    for t in range(NS):
        trow = cbuf[pl.ds(t * 16, 16)]
        pref = pref + jnp.where(jnp.int32(t) < s, trow, 0)
        tot = tot + trow
    for lb in range(2):
        gb = 2 * c + lb
        thr_b = _lane(thrvec, gb, iota)
        gt_pref = _lane(pref, 2 * lb, iota)
        eq_pref = _lane(pref, 2 * lb + 1, iota)
        gt_all = _lane(tot, 2 * lb, iota)
        ties = jnp.int32(K) - gt_all
        lg = jnp.int32(0)
        le = jnp.int32(0)
        for ch in range(TOK // 16):
            sv = sbuf[pl.ds(lb * TOK + ch * 16, 16)]
            kv = _sortkey(plsc.bitcast(sv, jnp.int32))
            gt = kv > thr_b
            eq = kv == thr_b
            gti = gt.astype(jnp.int32)
            eqi = eq.astype(jnp.int32)
            gex = plsc.cumsum(gti) - gti + lg
            eex = plsc.cumsum(eqi) - eqi + le
            lg = lg + jnp.sum(gti)
            le = le + jnp.sum(eqi)
            gbe = gt_pref + gex
            ebe = eq_pref + eex
            sel = jnp.logical_or(gt, jnp.logical_and(eq, ebe < ties))
            pos = gbe + jnp.minimum(ebe, ties)
            rowv = gb * S + s * TOK + ch * 16 + iota
            posf = jnp.where(sel, lb * IDXPAD + pos,
                             jnp.int32(lb * IDXPAD + K) + iota)
            pos2d[ch] = posf
            vals2d[ch] = rowv
        handles = [pltpu.async_copy(vals2d.at[ch], idx_sh.at[pos2d.at[ch]], sem)
                   for ch in range(TOK // 16)]
        for hnd in handles:
            hnd.wait()
    plsc.subcore_barrier()

    for lb in range(2):
        gb = 2 * c + lb
        pltpu.sync_copy(idx_sh.at[pl.ds(lb * IDXPAD + s * OUTT, OUTT)], idxg)
        out_base = gb * K + s * OUTT
        for j in range(OUTT // 16):
            pltpu.async_copy(
                x_hbm.at[idxg.at[pl.ds(j * 16, 16)]], rowbuf, semg).wait()
            pltpu.sync_copy(rowbuf, out_hbm.at[pl.ds(out_base + j * 16, 16)])


def _compact_gather(x2d, scores_flat, thr16):
    mesh = plsc.VectorSubcoreMesh(core_axis_name="c", subcore_axis_name="s")
    kfn = pl.kernel(
        _sc_body,
        out_type=jax.ShapeDtypeStruct((B * K, H), jnp.float32),
        mesh=mesh,
        compiler_params=pltpu.CompilerParams(needs_layout_passes=False),
        scratch_types=[
            pltpu.VMEM((2 * TOK,), jnp.float32),
            pltpu.VMEM((16,), jnp.int32),
            pltpu.VMEM((NS * 16,), jnp.int32),
            pltpu.VMEM((16,), jnp.int32),
            pltpu.VMEM((TOK // 16, 16), jnp.int32),
            pltpu.VMEM((TOK // 16, 16), jnp.int32),
            pltpu.VMEM((OUTT,), jnp.int32),
            pltpu.VMEM((16, H), jnp.float32),
            pltpu.VMEM_SHARED((2 * IDXPAD,), jnp.int32),
            pltpu.VMEM_SHARED((NS * 16,), jnp.int32),
            pltpu.SemaphoreType.DMA,
            pltpu.SemaphoreType.DMA,
        ],
    )
    return kfn(x2d, scores_flat, thr16)


def kernel(x, W1, b1, W2, b2):
    x2d = x.reshape(NROWS, H)
    scores = _scores(x2d, W1, b1, W2, b2)
    scores4 = scores.reshape(B, S)
    thr = _threshold(scores4)
    thr16 = jnp.concatenate(
        [thr[:, 0], jnp.zeros((16 - B,), jnp.int32)])
    out2d = _compact_gather(x2d, scores.reshape(NROWS), thr16)
    return out2d.reshape(B, K, H)


# speedup vs baseline: 1.5042x; 1.1161x over previous
import functools

import jax
import jax.numpy as jnp
from jax import lax
from jax.experimental import pallas as pl
from jax.experimental.pallas import tpu as pltpu
from jax.experimental.pallas import tpu_sc as plsc

B, S, H, D = 4, 8192, 1024, 256
K = S // 2
RB = 1024
NROWS = B * S
NS = 16
TOK = S // NS
OUTT = K // NS
IDXPAD = 4224

_SQRT_HALF = 0.7071067811865476
_MAXLOG = 88.72283905206835
_ERFC_P = (2.326819970068386e-2, -1.387039388740657e-1, 3.687424674597105e-1,
           -5.824733027278666e-1, 6.210004621745983e-1, -4.944515323274145e-1,
           3.404879937665872e-1, -2.741127028184656e-1, 5.638259427386472e-1)
_ERFC_R = (-1.047766399936249e+1, 1.297719955372516e+1, -7.495518717768503e+0,
           2.921019019210786e+0, -1.015265279202700e+0, 4.218463358204948e-1,
           -2.820767439740514e-1, 5.641895067754075e-1)
_ERF_T = (7.853861353153693e-5, -8.010193625184903e-4, 5.188327685732524e-3,
          -2.685381193529856e-2, 1.128358514861418e-1, -3.761262582423300e-1,
          1.128379165726710e+0)


def _poly(y, coeffs):
    p = jnp.zeros_like(y)
    for c in coeffs:
        p = p * y + jnp.float32(c)
    return p


def _erfc_f32(x):
    one, two, zero = jnp.float32(1), jnp.float32(2), jnp.float32(0)
    x2 = x * x
    z = -x2
    absx = jnp.abs(x)
    rabs = one / absx
    y = rabs * rabs
    ez = jnp.exp(z)
    poll = jnp.where(absx < two, _poly(y, _ERFC_P), _poly(y, _ERFC_R))
    erfc_ge = (ez * rabs) * poll
    erfc_ge = jnp.where(z < jnp.float32(-_MAXLOG), zero, erfc_ge)
    erfc_ge = jnp.where(x < zero, two - erfc_ge, erfc_ge)
    erf_small = x * _poly(x2, _ERF_T)
    return jnp.where(absx < one, one - erf_small, erfc_ge)


def _score_body(x_ref, w1_ref, b1_ref, w2_ref, b2_ref, out_ref):
    xb = x_ref[...]
    h_pre = lax.dot_general(
        xb, w1_ref[...], (((1,), (0,)), ((), ())),
        preferred_element_type=jnp.float32)
    h_pre = h_pre + b1_ref[...]
    g = (jnp.float32(0.5) * h_pre) * _erfc_f32(-h_pre * jnp.float32(_SQRT_HALF))
    imp = lax.dot_general(
        g, w2_ref[...], (((1,), (0,)), ((), ())),
        preferred_element_type=jnp.float32) + b2_ref[0, 0]
    out_ref[...] = imp


def _scores(x2d, W1, b1, W2, b2):
    return pl.pallas_call(
        _score_body,
        grid=(NROWS // RB,),
        in_specs=[
            pl.BlockSpec((RB, H), lambda i: (i, 0)),
            pl.BlockSpec((H, D), lambda i: (0, 0)),
            pl.BlockSpec((1, D), lambda i: (0, 0)),
            pl.BlockSpec((D, 1), lambda i: (0, 0)),
            pl.BlockSpec(memory_space=pltpu.SMEM),
        ],
        out_specs=pl.BlockSpec((RB, 1), lambda i: (i, 0)),
        out_shape=jax.ShapeDtypeStruct((NROWS, 1), jnp.float32),
        compiler_params=pltpu.CompilerParams(
            vmem_limit_bytes=100 * 1024 * 1024),
    )(x2d, W1, b1.reshape(1, D), W2, b2.reshape(1, 1))


def _sortkey(b):
    return b ^ ((b >> 31) & jnp.int32(0x7FFFFFFF))


def _thr_body(s_ref, out_ref):
    keys = _sortkey(lax.bitcast_convert_type(s_ref[...], jnp.int32))
    kk = jnp.int32(K)
    isign = jnp.int32(-2147483648)

    def body(t, vu):
        bit = 31 - t
        cand_u = vu | (jnp.int32(1) << bit)
        cand_s = cand_u ^ isign
        cnt = jnp.sum((keys >= cand_s).astype(jnp.int32), axis=1, keepdims=True)
        return jnp.where(cnt >= kk, cand_u, vu)

    vu = lax.fori_loop(0, 32, body, jnp.zeros((B, 1), jnp.int32))
    vs = vu ^ isign
    out_ref[...] = jnp.broadcast_to(vs, (B, 128))


def _threshold(scores4):
    return pl.pallas_call(
        _thr_body,
        in_specs=[pl.BlockSpec((B, S), lambda: (0, 0))],
        out_specs=pl.BlockSpec((B, 128), lambda: (0, 0)),
        out_shape=jax.ShapeDtypeStruct((B, 128), jnp.int32),
    )(scores4)


def _lane(v, i, iota):
    return jnp.sum(jnp.where(iota == i, v, 0))


def _sc_body(x_hbm, sc_hbm, thr_hbm, out_hbm,
             sbuf, thrv, cbuf, cntrow, pos2d, vals2d, idxg,
             rowbuf0, rowbuf1, rowbuf2,
             idx_sh, cnt_sh, sem,
             gsem0, gsem1, gsem2, wsem0, wsem1, wsem2):
    rowbufs = (rowbuf0, rowbuf1, rowbuf2)
    gsems = (gsem0, gsem1, gsem2)
    wsems = (wsem0, wsem1, wsem2)
    c = lax.axis_index("c")
    s = lax.axis_index("s")
    iota = lax.iota(jnp.int32, 16)
    pltpu.sync_copy(thr_hbm, thrv)
    thrvec = thrv[...]

    cnts = []
    for lb in range(2):
        gb = 2 * c + lb
        off = gb * S + s * TOK
        pltpu.sync_copy(sc_hbm.at[pl.ds(off, TOK)], sbuf.at[pl.ds(lb * TOK, TOK)])
        thr_b = _lane(thrvec, gb, iota)
        gcnt = jnp.int32(0)
        ecnt = jnp.int32(0)
        for ch in range(TOK // 16):
            sv = sbuf[pl.ds(lb * TOK + ch * 16, 16)]
            kv = _sortkey(plsc.bitcast(sv, jnp.int32))
            gcnt = gcnt + jnp.sum((kv > thr_b).astype(jnp.int32))
            ecnt = ecnt + jnp.sum((kv == thr_b).astype(jnp.int32))
        cnts += [gcnt, ecnt]
    row = (jnp.where(iota == 0, cnts[0], 0) + jnp.where(iota == 1, cnts[1], 0)
           + jnp.where(iota == 2, cnts[2], 0) + jnp.where(iota == 3, cnts[3], 0))
    cntrow[...] = row
    pltpu.sync_copy(cntrow, cnt_sh.at[pl.ds(s * 16, 16)])
    plsc.subcore_barrier()

    pltpu.sync_copy(cnt_sh, cbuf)
    pref = jnp.zeros((16,), jnp.int32)
    tot = jnp.zeros((16,), jnp.int32)
    for t in range(NS):
        trow = cbuf[pl.ds(t * 16, 16)]
        pref = pref + jnp.where(jnp.int32(t) < s, trow, 0)
        tot = tot + trow
    for lb in range(2):
        gb = 2 * c + lb
        thr_b = _lane(thrvec, gb, iota)
        gt_pref = _lane(pref, 2 * lb, iota)
        eq_pref = _lane(pref, 2 * lb + 1, iota)
        gt_all = _lane(tot, 2 * lb, iota)
        ties = jnp.int32(K) - gt_all
        lg = jnp.int32(0)
        le = jnp.int32(0)
        for ch in range(TOK // 16):
            sv = sbuf[pl.ds(lb * TOK + ch * 16, 16)]
            kv = _sortkey(plsc.bitcast(sv, jnp.int32))
            gt = kv > thr_b
            eq = kv == thr_b
            gti = gt.astype(jnp.int32)
            eqi = eq.astype(jnp.int32)
            gex = plsc.cumsum(gti) - gti + lg
            eex = plsc.cumsum(eqi) - eqi + le
            lg = lg + jnp.sum(gti)
            le = le + jnp.sum(eqi)
            gbe = gt_pref + gex
            ebe = eq_pref + eex
            sel = jnp.logical_or(gt, jnp.logical_and(eq, ebe < ties))
            pos = gbe + jnp.minimum(ebe, ties)
            rowv = gb * S + s * TOK + ch * 16 + iota
            posf = jnp.where(sel, lb * IDXPAD + pos,
                             jnp.int32(lb * IDXPAD + K) + iota)
            pos2d[ch] = posf
            vals2d[ch] = rowv
        handles = [pltpu.async_copy(vals2d.at[ch], idx_sh.at[pos2d.at[ch]], sem)
                   for ch in range(TOK // 16)]
        for hnd in handles:
            hnd.wait()
    plsc.subcore_barrier()

    for lb in range(2):
        gb = 2 * c + lb
        pltpu.sync_copy(idx_sh.at[pl.ds(lb * IDXPAD + s * OUTT, OUTT)], idxg)
        out_base = gb * K + s * OUTT
        nch = OUTT // 16
        gh = {}
        wh = {}
        for j in range(nch):
            kb = j % 3
            if j >= 3:
                wh[j - 3].wait()
            gh[j] = pltpu.async_copy(
                x_hbm.at[idxg.at[pl.ds(j * 16, 16)]], rowbufs[kb], gsems[kb])
            if j >= 1:
                gh[j - 1].wait()
                wh[j - 1] = pltpu.async_copy(
                    rowbufs[(j - 1) % 3],
                    out_hbm.at[pl.ds(out_base + (j - 1) * 16, 16)],
                    wsems[(j - 1) % 3])
        gh[nch - 1].wait()
        wh[nch - 1] = pltpu.async_copy(
            rowbufs[(nch - 1) % 3],
            out_hbm.at[pl.ds(out_base + (nch - 1) * 16, 16)],
            wsems[(nch - 1) % 3])
        for j in range(nch - 3, nch):
            wh[j].wait()


def _compact_gather(x2d, scores_flat, thr16):
    mesh = plsc.VectorSubcoreMesh(core_axis_name="c", subcore_axis_name="s")
    kfn = pl.kernel(
        _sc_body,
        out_type=jax.ShapeDtypeStruct((B * K, H), jnp.float32),
        mesh=mesh,
        compiler_params=pltpu.CompilerParams(needs_layout_passes=False),
        scratch_types=[
            pltpu.VMEM((2 * TOK,), jnp.float32),
            pltpu.VMEM((16,), jnp.int32),
            pltpu.VMEM((NS * 16,), jnp.int32),
            pltpu.VMEM((16,), jnp.int32),
            pltpu.VMEM((TOK // 16, 16), jnp.int32),
            pltpu.VMEM((TOK // 16, 16), jnp.int32),
            pltpu.VMEM((OUTT,), jnp.int32),
            pltpu.VMEM((16, H), jnp.float32),
            pltpu.VMEM((16, H), jnp.float32),
            pltpu.VMEM((16, H), jnp.float32),
            pltpu.VMEM_SHARED((2 * IDXPAD,), jnp.int32),
            pltpu.VMEM_SHARED((NS * 16,), jnp.int32),
            pltpu.SemaphoreType.DMA,
            pltpu.SemaphoreType.DMA,
            pltpu.SemaphoreType.DMA,
            pltpu.SemaphoreType.DMA,
            pltpu.SemaphoreType.DMA,
            pltpu.SemaphoreType.DMA,
            pltpu.SemaphoreType.DMA,
        ],
    )
    return kfn(x2d, scores_flat, thr16)


def kernel(x, W1, b1, W2, b2):
    x2d = x.reshape(NROWS, H)
    scores = _scores(x2d, W1, b1, W2, b2)
    scores4 = scores.reshape(B, S)
    thr = _threshold(scores4)
    thr16 = jnp.concatenate(
        [thr[:, 0], jnp.zeros((16 - B,), jnp.int32)])
    out2d = _compact_gather(x2d, scores.reshape(NROWS), thr16)
    return out2d.reshape(B, K, H)
